# baseline (speedup 1.0000x reference)
import functools
import math

import jax
import jax.numpy as jnp
from jax import lax
from jax.experimental import pallas as pl
from jax.experimental.pallas import tpu as pltpu
from jax.experimental.pallas import tpu_sc as plsc

_IN_F = 4096
_OUT_F = 64
_K = 8
_NEG_INF = float("-inf")
_N_WORKERS = 32
_LANES = 16


def _matmul_body(x_ref, wt_ref, b_ref, out_ref):
    out_ref[...] = jax.lax.dot_general(
        x_ref[...], wt_ref[...], (((1,), (0,)), ((), ())),
        preferred_element_type=jnp.float32,
    ) + b_ref[...]


@functools.partial(jax.jit, static_argnames=("block",))
def _logits_tc(x, wt, b2d, block=1024):
    n = x.shape[0]
    return pl.pallas_call(
        _matmul_body,
        grid=(n // block,),
        in_specs=[
            pl.BlockSpec((block, _IN_F), lambda i: (i, 0)),
            pl.BlockSpec((_IN_F, _OUT_F), lambda i: (0, 0)),
            pl.BlockSpec((1, _OUT_F), lambda i: (0, 0)),
        ],
        out_specs=pl.BlockSpec((block, _OUT_F), lambda i: (i, 0)),
        out_shape=jax.ShapeDtypeStruct((n, _OUT_F), jnp.float32),
        compiler_params=pltpu.CompilerParams(
            dimension_semantics=("arbitrary",),
        ),
    )(x, wt, b2d)


def _topk_sc_body(logits_hbm, vals_hbm, idx_hbm, buf, vals_v, idx_v):
    n_tok = logits_hbm.shape[0] // _OUT_F
    t_per_w = n_tok // _N_WORKERS
    n_groups = t_per_w // _LANES
    wid = lax.axis_index("c") * 16 + lax.axis_index("s")
    base = wid * t_per_w
    pltpu.sync_copy(logits_hbm.at[pl.ds(base * _OUT_F, t_per_w * _OUT_F)], buf)

    lane = lax.broadcasted_iota(jnp.int32, (_LANES,), 0)
    neg_inf_v = jnp.full((_LANES,), _NEG_INF, jnp.float32)

    def group_body(g, carry):
        row = (g * _LANES + lane) * _OUT_F
        out_row = (g * _LANES + lane) * _K

        def pass_body(k, carry2):
            m = plsc.load_gather(buf, [row])
            mi = jnp.zeros((_LANES,), jnp.int32)
            for e in range(1, _OUT_F):
                v = plsc.load_gather(buf, [row + e])
                gt = v > m
                m = jnp.where(gt, v, m)
                mi = jnp.where(gt, jnp.full((_LANES,), e, jnp.int32), mi)
            plsc.store_scatter(buf, [row + mi], neg_inf_v)
            plsc.store_scatter(vals_v, [out_row + k], m)
            plsc.store_scatter(idx_v, [out_row + k], mi)
            return carry2

        return lax.fori_loop(0, _K, pass_body, carry)

    lax.fori_loop(0, n_groups, group_body, 0)
    pltpu.sync_copy(vals_v, vals_hbm.at[pl.ds(base * _K, t_per_w * _K)])
    pltpu.sync_copy(idx_v, idx_hbm.at[pl.ds(base * _K, t_per_w * _K)])


@jax.jit
def _topk_sc(logits):
    n = logits.shape[0]
    t_per_w = n // _N_WORKERS
    mesh = plsc.VectorSubcoreMesh(core_axis_name="c", subcore_axis_name="s")
    f = functools.partial(
        pl.kernel,
        out_type=[
            jax.ShapeDtypeStruct((n * _K,), jnp.float32),
            jax.ShapeDtypeStruct((n * _K,), jnp.int32),
        ],
        mesh=mesh,
        scratch_types=[
            pltpu.VMEM((t_per_w * _OUT_F,), jnp.float32),
            pltpu.VMEM((t_per_w * _K,), jnp.float32),
            pltpu.VMEM((t_per_w * _K,), jnp.int32),
        ],
        compiler_params=pltpu.CompilerParams(needs_layout_passes=False),
    )(_topk_sc_body)
    vals, idx = f(logits.reshape(-1))
    return vals.reshape(n, _K), idx.reshape(n, _K)


def kernel(input, weight, bias):
    wt = weight.T
    b2d = bias.reshape(1, _OUT_F)
    logits = _logits_tc(input, wt, b2d)
    vals, idx = _topk_sc(logits)
    return (vals, idx)


# speedup vs baseline: 1.0637x; 1.0637x over previous
import functools
import math

import jax
import jax.numpy as jnp
from jax import lax
from jax.experimental import pallas as pl
from jax.experimental.pallas import tpu as pltpu
from jax.experimental.pallas import tpu_sc as plsc

_IN_F = 4096
_OUT_F = 64
_K = 8
_NEG_INF = float("-inf")
_N_WORKERS = 32
_LANES = 16


def _matmul_body(x_ref, wt_ref, b_ref, out_ref):
    out_ref[...] = jax.lax.dot_general(
        x_ref[...], wt_ref[...], (((1,), (0,)), ((), ())),
        preferred_element_type=jnp.float32,
    ) + b_ref[...]


@functools.partial(jax.jit, static_argnames=("block",))
def _logits_tc(x, wt, b2d, block=1024):
    n = x.shape[0]
    return pl.pallas_call(
        _matmul_body,
        grid=(n // block,),
        in_specs=[
            pl.BlockSpec((block, _IN_F), lambda i: (i, 0)),
            pl.BlockSpec((_IN_F, _OUT_F), lambda i: (0, 0)),
            pl.BlockSpec((1, _OUT_F), lambda i: (0, 0)),
        ],
        out_specs=pl.BlockSpec((block, _OUT_F), lambda i: (i, 0)),
        out_shape=jax.ShapeDtypeStruct((n, _OUT_F), jnp.float32),
        compiler_params=pltpu.CompilerParams(
            dimension_semantics=("arbitrary",),
        ),
    )(x, wt, b2d)


def _topk_sc_body(logits_hbm, vals_hbm, idx_hbm, buf, vals_v, idx_v):
    n_tok = logits_hbm.shape[0] // _OUT_F
    t_per_w = n_tok // _N_WORKERS
    n_groups = t_per_w // _LANES
    wid = lax.axis_index("c") * 16 + lax.axis_index("s")
    base = wid * t_per_w
    pltpu.sync_copy(logits_hbm.at[pl.ds(base * _OUT_F, t_per_w * _OUT_F)], buf)

    lane = lax.broadcasted_iota(jnp.int32, (_LANES,), 0)
    neg_inf_v = jnp.full((_LANES,), _NEG_INF, jnp.float32)

    def group_body(g, carry):
        row = (g * _LANES + lane) * _OUT_F
        out_row = (g * _LANES + lane) * _K

        def pass_body(k, carry2):
            vs = [plsc.load_gather(buf, [row + e]) for e in range(_OUT_F)]
            is_ = [jnp.full((_LANES,), e, jnp.int32) for e in range(_OUT_F)]
            while len(vs) > 1:
                nvs, nis = [], []
                for j in range(0, len(vs), 2):
                    keep = vs[j] >= vs[j + 1]
                    nvs.append(jnp.where(keep, vs[j], vs[j + 1]))
                    nis.append(jnp.where(keep, is_[j], is_[j + 1]))
                vs, is_ = nvs, nis
            m, mi = vs[0], is_[0]
            plsc.store_scatter(buf, [row + mi], neg_inf_v)
            plsc.store_scatter(vals_v, [out_row + k], m)
            plsc.store_scatter(idx_v, [out_row + k], mi)
            return carry2

        return lax.fori_loop(0, _K, pass_body, carry)

    lax.fori_loop(0, n_groups, group_body, 0)
    pltpu.sync_copy(vals_v, vals_hbm.at[pl.ds(base * _K, t_per_w * _K)])
    pltpu.sync_copy(idx_v, idx_hbm.at[pl.ds(base * _K, t_per_w * _K)])


@jax.jit
def _topk_sc(logits):
    n = logits.shape[0]
    t_per_w = n // _N_WORKERS
    mesh = plsc.VectorSubcoreMesh(core_axis_name="c", subcore_axis_name="s")
    f = functools.partial(
        pl.kernel,
        out_type=[
            jax.ShapeDtypeStruct((n * _K,), jnp.float32),
            jax.ShapeDtypeStruct((n * _K,), jnp.int32),
        ],
        mesh=mesh,
        scratch_types=[
            pltpu.VMEM((t_per_w * _OUT_F,), jnp.float32),
            pltpu.VMEM((t_per_w * _K,), jnp.float32),
            pltpu.VMEM((t_per_w * _K,), jnp.int32),
        ],
        compiler_params=pltpu.CompilerParams(needs_layout_passes=False),
    )(_topk_sc_body)
    vals, idx = f(logits.reshape(-1))
    return vals.reshape(n, _K), idx.reshape(n, _K)


def kernel(input, weight, bias):
    wt = weight.T
    b2d = bias.reshape(1, _OUT_F)
    logits = _logits_tc(input, wt, b2d)
    vals, idx = _topk_sc(logits)
    return (vals, idx)


# speedup vs baseline: 1.6665x; 1.5667x over previous
import functools
import math

import jax
import jax.numpy as jnp
from jax import lax
from jax.experimental import pallas as pl
from jax.experimental.pallas import tpu as pltpu
from jax.experimental.pallas import tpu_sc as plsc

_IN_F = 4096
_OUT_F = 64
_K = 8
_NEG_INF = float("-inf")
_N_WORKERS = 32
_LANES = 16


_STRIDE = 65


def _matmul_body(x_ref, wt_ref, b_ref, out_ref):
    logits = jax.lax.dot_general(
        x_ref[...], wt_ref[...], (((1,), (0,)), ((), ())),
        preferred_element_type=jnp.float32,
    ) + b_ref[...]
    pad = jnp.zeros((logits.shape[0], _STRIDE - _OUT_F), jnp.float32)
    out_ref[...] = jnp.concatenate([logits, pad], axis=1)


@functools.partial(jax.jit, static_argnames=("block",))
def _logits_tc(x, wt, b2d, block=1024):
    n = x.shape[0]
    return pl.pallas_call(
        _matmul_body,
        grid=(n // block,),
        in_specs=[
            pl.BlockSpec((block, _IN_F), lambda i: (i, 0)),
            pl.BlockSpec((_IN_F, _OUT_F), lambda i: (0, 0)),
            pl.BlockSpec((1, _OUT_F), lambda i: (0, 0)),
        ],
        out_specs=pl.BlockSpec((block, _STRIDE), lambda i: (i, 0)),
        out_shape=jax.ShapeDtypeStruct((n, _STRIDE), jnp.float32),
        compiler_params=pltpu.CompilerParams(
            dimension_semantics=("arbitrary",),
        ),
    )(x, wt, b2d)


def _topk_sc_body(logits_hbm, vals_hbm, idx_hbm, buf, vals_v, idx_v):
    n_tok = logits_hbm.shape[0] // _STRIDE
    t_per_w = n_tok // _N_WORKERS
    n_groups = t_per_w // _LANES
    wid = lax.axis_index("c") * 16 + lax.axis_index("s")
    base = wid * t_per_w
    pltpu.sync_copy(logits_hbm.at[pl.ds(base * _STRIDE, t_per_w * _STRIDE)], buf)

    lane = lax.broadcasted_iota(jnp.int32, (_LANES,), 0)
    neg_inf_v = jnp.full((_LANES,), _NEG_INF, jnp.float32)

    def group_body(g, carry):
        row = (g * _LANES + lane) * _STRIDE
        out_row = (g * _LANES + lane) * _K

        def pass_body(k, carry2):
            vs = [plsc.load_gather(buf, [row + e]) for e in range(_OUT_F)]
            is_ = [jnp.full((_LANES,), e, jnp.int32) for e in range(_OUT_F)]
            while len(vs) > 1:
                nvs, nis = [], []
                for j in range(0, len(vs), 2):
                    keep = vs[j] >= vs[j + 1]
                    nvs.append(jnp.where(keep, vs[j], vs[j + 1]))
                    nis.append(jnp.where(keep, is_[j], is_[j + 1]))
                vs, is_ = nvs, nis
            m, mi = vs[0], is_[0]
            plsc.store_scatter(buf, [row + mi], neg_inf_v)
            plsc.store_scatter(vals_v, [out_row + k], m)
            plsc.store_scatter(idx_v, [out_row + k], mi)
            return carry2

        return lax.fori_loop(0, _K, pass_body, carry)

    lax.fori_loop(0, n_groups, group_body, 0)
    pltpu.sync_copy(vals_v, vals_hbm.at[pl.ds(base * _K, t_per_w * _K)])
    pltpu.sync_copy(idx_v, idx_hbm.at[pl.ds(base * _K, t_per_w * _K)])


@jax.jit
def _topk_sc(logits):
    n = logits.shape[0]
    t_per_w = n // _N_WORKERS
    mesh = plsc.VectorSubcoreMesh(core_axis_name="c", subcore_axis_name="s")
    f = functools.partial(
        pl.kernel,
        out_type=[
            jax.ShapeDtypeStruct((n * _K,), jnp.float32),
            jax.ShapeDtypeStruct((n * _K,), jnp.int32),
        ],
        mesh=mesh,
        scratch_types=[
            pltpu.VMEM((t_per_w * _STRIDE,), jnp.float32),
            pltpu.VMEM((t_per_w * _K,), jnp.float32),
            pltpu.VMEM((t_per_w * _K,), jnp.int32),
        ],
        compiler_params=pltpu.CompilerParams(needs_layout_passes=False),
    )(_topk_sc_body)
    vals, idx = f(logits.reshape(-1))
    return vals.reshape(n, _K), idx.reshape(n, _K)


def kernel(input, weight, bias):
    wt = weight.T
    b2d = bias.reshape(1, _OUT_F)
    logits = _logits_tc(input, wt, b2d)
    vals, idx = _topk_sc(logits)
    return (vals, idx)


# speedup vs baseline: 1.7733x; 1.0641x over previous
import functools
import math

import jax
import jax.numpy as jnp
from jax import lax
from jax.experimental import pallas as pl
from jax.experimental.pallas import tpu as pltpu
from jax.experimental.pallas import tpu_sc as plsc

_IN_F = 4096
_OUT_F = 64
_K = 8
_NEG_INF = float("-inf")
_N_WORKERS = 32
_LANES = 16


_STRIDE = 65


def _matmul_body(x_ref, wt_ref, b_ref, out_ref):
    logits = jax.lax.dot_general(
        x_ref[...], wt_ref[...], (((1,), (0,)), ((), ())),
        preferred_element_type=jnp.float32,
    ) + b_ref[...]
    pad = jnp.zeros((logits.shape[0], _STRIDE - _OUT_F), jnp.float32)
    out_ref[...] = jnp.concatenate([logits, pad], axis=1)


@functools.partial(jax.jit, static_argnames=("block",))
def _logits_tc(x, wt, b2d, block=1024):
    n = x.shape[0]
    return pl.pallas_call(
        _matmul_body,
        grid=(n // block,),
        in_specs=[
            pl.BlockSpec((block, _IN_F), lambda i: (i, 0)),
            pl.BlockSpec((_IN_F, _OUT_F), lambda i: (0, 0)),
            pl.BlockSpec((1, _OUT_F), lambda i: (0, 0)),
        ],
        out_specs=pl.BlockSpec((block, _STRIDE), lambda i: (i, 0)),
        out_shape=jax.ShapeDtypeStruct((n, _STRIDE), jnp.float32),
        compiler_params=pltpu.CompilerParams(
            dimension_semantics=("arbitrary",),
        ),
    )(x, wt, b2d)


def _topk_sc_body(logits_hbm, vals_hbm, idx_hbm, buf, vals_v, idx_v):
    n_tok = logits_hbm.shape[0] // _STRIDE
    t_per_w = n_tok // _N_WORKERS
    n_groups = t_per_w // _LANES
    wid = lax.axis_index("c") * 16 + lax.axis_index("s")
    base = wid * t_per_w
    pltpu.sync_copy(logits_hbm.at[pl.ds(base * _STRIDE, t_per_w * _STRIDE)], buf)

    lane = lax.broadcasted_iota(jnp.int32, (_LANES,), 0)
    neg_inf_v = jnp.full((_LANES,), _NEG_INF, jnp.float32)

    def _tree(vs, is_):
        while len(vs) > 1:
            nvs, nis = [], []
            for j in range(0, len(vs), 2):
                keep = vs[j] >= vs[j + 1]
                nvs.append(jnp.where(keep, vs[j], vs[j + 1]))
                nis.append(jnp.where(keep, is_[j], is_[j + 1]))
            vs, is_ = nvs, nis
        return vs[0], is_[0]

    n_chunks = _OUT_F // _LANES

    def group_body(g, carry):
        row = (g * _LANES + lane) * _STRIDE
        out_row = (g * _LANES + lane) * _K

        cvs, cis = [], []
        for c in range(n_chunks):
            vs = [plsc.load_gather(buf, [row + (c * _LANES + j)])
                  for j in range(_LANES)]
            is_ = [jnp.full((_LANES,), c * _LANES + j, jnp.int32)
                   for j in range(_LANES)]
            cv, ci = _tree(vs, is_)
            cvs.append(cv)
            cis.append(ci)

        def pass_body(k, carry2):
            cv_l = list(carry2[:n_chunks])
            ci_l = list(carry2[n_chunks:])
            m, mi = _tree(list(cv_l), list(ci_l))
            plsc.store_scatter(vals_v, [out_row + k], m)
            plsc.store_scatter(idx_v, [out_row + k], mi)
            plsc.store_scatter(buf, [row + mi], neg_inf_v)
            cb = jnp.bitwise_and(mi, jnp.full((_LANES,), -_LANES, jnp.int32))
            vs = [plsc.load_gather(buf, [row + cb + j]) for j in range(_LANES)]
            is_ = [cb + j for j in range(_LANES)]
            nv, ni = _tree(vs, is_)
            cid = jnp.right_shift(mi, 4)
            for c in range(n_chunks):
                hit = cid == c
                cv_l[c] = jnp.where(hit, nv, cv_l[c])
                ci_l[c] = jnp.where(hit, ni, ci_l[c])
            return tuple(cv_l) + tuple(ci_l)

        lax.fori_loop(0, _K, pass_body, tuple(cvs) + tuple(cis))
        return carry

    lax.fori_loop(0, n_groups, group_body, 0)
    pltpu.sync_copy(vals_v, vals_hbm.at[pl.ds(base * _K, t_per_w * _K)])
    pltpu.sync_copy(idx_v, idx_hbm.at[pl.ds(base * _K, t_per_w * _K)])


@jax.jit
def _topk_sc(logits):
    n = logits.shape[0]
    t_per_w = n // _N_WORKERS
    mesh = plsc.VectorSubcoreMesh(core_axis_name="c", subcore_axis_name="s")
    f = functools.partial(
        pl.kernel,
        out_type=[
            jax.ShapeDtypeStruct((n * _K,), jnp.float32),
            jax.ShapeDtypeStruct((n * _K,), jnp.int32),
        ],
        mesh=mesh,
        scratch_types=[
            pltpu.VMEM((t_per_w * _STRIDE,), jnp.float32),
            pltpu.VMEM((t_per_w * _K,), jnp.float32),
            pltpu.VMEM((t_per_w * _K,), jnp.int32),
        ],
        compiler_params=pltpu.CompilerParams(needs_layout_passes=False),
    )(_topk_sc_body)
    vals, idx = f(logits.reshape(-1))
    return vals.reshape(n, _K), idx.reshape(n, _K)


def kernel(input, weight, bias):
    wt = weight.T
    b2d = bias.reshape(1, _OUT_F)
    logits = _logits_tc(input, wt, b2d)
    vals, idx = _topk_sc(logits)
    return (vals, idx)
